# initial kernel scaffold (unmeasured)
import jax
import jax.numpy as jnp
from jax import lax
from jax.experimental import pallas as pl
from jax.experimental.pallas import tpu as pltpu

B, H, D, BS = 8, 8, 128, 16
NB = 512
Y = 4
NP_LOCAL = 2048 // Y
NKEYS = NP_LOCAL * BS
SCALE = D ** -0.5

PAY = 3 * D


def kernel(Q, K, V, bt, lens):
    def body(q_ref, k_ref, v_ref, bt_ref, lens_ref, out_ref,
             comm_ref, send_sems, recv_sems):
        my_x = lax.axis_index("x")
        my_y = lax.axis_index("y")
        my_z = lax.axis_index("z")
        left = (my_y - 1) % Y
        right = (my_y + 1) % Y

        barrier_sem = pltpu.get_barrier_semaphore()
        for nbr in (left, right):
            pl.semaphore_signal(
                barrier_sem, inc=1,
                device_id=(my_x, nbr, my_z),
                device_id_type=pl.DeviceIdType.MESH,
            )
        pl.semaphore_wait(barrier_sem, 2)

        qs = q_ref[:, 0, :, :]
        k2 = k_ref[:].reshape(NKEYS, H, D)
        v2 = v_ref[:].reshape(NKEYS, H, D)
        btv = bt_ref[:]
        lens_v = lens_ref[:]

        j_iota = lax.broadcasted_iota(jnp.int32, (B, NB, NP_LOCAL), 1)
        p_iota = lax.broadcasted_iota(jnp.int32, (B, NB, NP_LOCAL), 2)
        hit = (btv[:, :, None] == my_y * NP_LOCAL + p_iota) & (
            j_iota < lens_v[:, :, None]
        )
        c = jnp.sum(hit.astype(jnp.float32), axis=1)
        cw = jnp.broadcast_to(c[:, :, None], (B, NP_LOCAL, BS))
        cw = cw.reshape(B, NKEYS)

        s = lax.dot_general(
            qs.astype(jnp.bfloat16), k2.astype(jnp.bfloat16),
            dimension_numbers=(((2,), (2,)), ((1,), (1,))),
            preferred_element_type=jnp.float32,
        ) * SCALE
        smask = jnp.where(cw[None, :, :] > 0, s, -1e9)
        m = jnp.max(smask, axis=-1)
        e = jnp.exp(smask - m[:, :, None]) * cw[None, :, :]
        l = jnp.sum(e, axis=-1)
        acc = lax.dot_general(
            e.astype(jnp.bfloat16), v2.astype(jnp.bfloat16),
            dimension_numbers=(((2,), (0,)), ((0,), (1,))),
            preferred_element_type=jnp.float32,
        )

        comm_ref[0, :, :, 0:D] = acc
        comm_ref[0, :, :, D:2 * D] = jnp.broadcast_to(m[:, :, None], (H, B, D))
        comm_ref[0, :, :, 2 * D:3 * D] = jnp.broadcast_to(l[:, :, None], (H, B, D))

        for h in range(Y - 1):
            rdma = pltpu.make_async_remote_copy(
                src_ref=comm_ref.at[h],
                dst_ref=comm_ref.at[h + 1],
                send_sem=send_sems.at[h],
                recv_sem=recv_sems.at[h],
                device_id=(my_x, right, my_z),
                device_id_type=pl.DeviceIdType.MESH,
            )
            rdma.start()
            rdma.wait()

        slots = [comm_ref[s] for s in range(Y)]
        ms = [a[:, :, D:2 * D] for a in slots]
        m_g = jnp.maximum(jnp.maximum(ms[0], ms[1]),
                          jnp.maximum(ms[2], ms[3]))
        acc_g = jnp.zeros((H, B, D), jnp.float32)
        l_g = jnp.zeros((H, B, D), jnp.float32)
        for a, mm in zip(slots, ms):
            sc = jnp.exp(mm - m_g)
            acc_g = acc_g + a[:, :, 0:D] * sc
            l_g = l_g + a[:, :, 2 * D:3 * D] * sc
        res = acc_g / l_g
        out_ref[:] = jnp.transpose(res, (1, 0, 2))[:, None, :, :]

    return pl.pallas_call(
        body,
        out_shape=jax.ShapeDtypeStruct((B, 1, H, D), jnp.float32),
        in_specs=[pl.BlockSpec(memory_space=pltpu.VMEM)] * 5,
        out_specs=pl.BlockSpec(memory_space=pltpu.VMEM),
        scratch_shapes=[
            pltpu.VMEM((Y, H, B, PAY), jnp.float32),
            pltpu.SemaphoreType.DMA((Y - 1,)),
            pltpu.SemaphoreType.DMA((Y - 1,)),
        ],
        compiler_params=pltpu.CompilerParams(collective_id=0),
    )(Q, K, V, bt, lens.reshape(B, 1))


# baseline (device time: 100987 ns/iter reference)
import jax
import jax.numpy as jnp
from jax import lax
from jax.experimental import pallas as pl
from jax.experimental.pallas import tpu as pltpu

B, H, D, BS = 8, 8, 128, 16
NB = 512
Y = 4
NP_LOCAL = 2048 // Y
CP = 64
NCHUNK = NP_LOCAL // CP
CKEYS = CP * BS
SCALE = D ** -0.5
NEG = -1e9

PAY = 3 * D


def _partial_kernel(Q, K, V, bt, lens):

    def body(q_ref, k_ref, v_ref, bt_ref, lens_ref, out_ref,
             kbuf, vbuf, ksems, vsems):
        my_y = lax.axis_index("y")

        qs = q_ref[:, 0, :, :].astype(jnp.bfloat16)
        btv = bt_ref[:]
        lens_v = lens_ref[:]
        j_valid = (
            lax.broadcasted_iota(jnp.int32, (B, NB, CP), 1)
            < lens_v[:, :, None]
        )
        p_iota = lax.broadcasted_iota(jnp.int32, (B, NB, CP), 2)

        def start_copy(ci, slot):
            pltpu.make_async_copy(
                k_ref.at[pl.ds(ci * CP, CP)], kbuf.at[slot], ksems.at[slot]
            ).start()
            pltpu.make_async_copy(
                v_ref.at[pl.ds(ci * CP, CP)], vbuf.at[slot], vsems.at[slot]
            ).start()

        def wait_copy(ci, slot):
            pltpu.make_async_copy(
                k_ref.at[pl.ds(ci * CP, CP)], kbuf.at[slot], ksems.at[slot]
            ).wait()
            pltpu.make_async_copy(
                v_ref.at[pl.ds(ci * CP, CP)], vbuf.at[slot], vsems.at[slot]
            ).wait()

        start_copy(0, 0)

        def chunk_body(ci, carry):
            m_run, l_run, acc_run = carry
            slot = lax.rem(ci, 2)

            @pl.when(ci + 1 < NCHUNK)
            def _():
                start_copy(ci + 1, lax.rem(ci + 1, 2))

            wait_copy(ci, slot)

            base = my_y * NP_LOCAL + ci * CP
            hit = (btv[:, :, None] == base + p_iota) & j_valid
            c = jnp.sum(hit.astype(jnp.float32), axis=1)
            cw = jnp.broadcast_to(c[:, :, None], (B, CP, BS))
            cw = cw.reshape(B, CKEYS)

            m_cols, l_cols, acc_cols = [], [], []
            for hI in range(H):
                kh = kbuf[slot, :, :, hI, :].reshape(CKEYS, D)
                vh = vbuf[slot, :, :, hI, :].reshape(CKEYS, D)
                qh = qs[:, hI, :]
                s_h = lax.dot_general(
                    qh, kh.astype(jnp.bfloat16),
                    dimension_numbers=(((1,), (1,)), ((), ())),
                    preferred_element_type=jnp.float32,
                ) * SCALE
                smask = jnp.where(cw > 0, s_h, NEG)
                m_prev = m_run[:, hI:hI + 1]
                m_new = jnp.maximum(
                    m_prev, jnp.max(smask, axis=-1, keepdims=True))
                alpha = jnp.exp(m_prev - m_new)
                e = jnp.exp(smask - m_new) * cw
                l_new = l_run[:, hI:hI + 1] * alpha + jnp.sum(
                    e, axis=-1, keepdims=True)
                pv = lax.dot_general(
                    e.astype(jnp.bfloat16), vh.astype(jnp.bfloat16),
                    dimension_numbers=(((1,), (0,)), ((), ())),
                    preferred_element_type=jnp.float32,
                )
                acc_cols.append(acc_run[:, hI, :] * alpha + pv)
                m_cols.append(m_new)
                l_cols.append(l_new)

            return (jnp.concatenate(m_cols, axis=1),
                    jnp.concatenate(l_cols, axis=1),
                    jnp.stack(acc_cols, axis=1))

        m_run, l_run, acc_run = lax.fori_loop(
            0, NCHUNK, chunk_body,
            (jnp.full((B, H), NEG, jnp.float32),
             jnp.zeros((B, H), jnp.float32),
             jnp.zeros((B, H, D), jnp.float32)),
        )

        out_ref[:, :, 0:D] = acc_run
        out_ref[:, :, D:2 * D] = jnp.broadcast_to(
            m_run[:, :, None], (B, H, D))
        out_ref[:, :, 2 * D:3 * D] = jnp.broadcast_to(
            l_run[:, :, None], (B, H, D))

    return pl.pallas_call(
        body,
        out_shape=jax.ShapeDtypeStruct((B, H, PAY), jnp.float32),
        in_specs=[
            pl.BlockSpec(memory_space=pltpu.VMEM),
            pl.BlockSpec(memory_space=pl.ANY),
            pl.BlockSpec(memory_space=pl.ANY),
            pl.BlockSpec(memory_space=pltpu.VMEM),
            pl.BlockSpec(memory_space=pltpu.VMEM),
        ],
        out_specs=pl.BlockSpec(memory_space=pltpu.VMEM),
        scratch_shapes=[
            pltpu.VMEM((2, CP, BS, H, D), jnp.float32),
            pltpu.VMEM((2, CP, BS, H, D), jnp.float32),
            pltpu.SemaphoreType.DMA((2,)),
            pltpu.SemaphoreType.DMA((2,)),
        ],
    )(Q, K, V, bt, lens.reshape(B, 1))


def _ring_combine_kernel(packed):

    def body(x_ref, out_ref, comm_ref, send_sems, recv_sems):
        my_x = lax.axis_index("x")
        my_y = lax.axis_index("y")
        my_z = lax.axis_index("z")
        left = (my_y - 1) % Y
        right = (my_y + 1) % Y

        barrier_sem = pltpu.get_barrier_semaphore()
        for nbr in (left, right):
            pl.semaphore_signal(
                barrier_sem, inc=1,
                device_id=(my_x, nbr, my_z),
                device_id_type=pl.DeviceIdType.MESH,
            )
        pl.semaphore_wait(barrier_sem, 2)

        comm_ref[0] = x_ref[:]

        for h in range(Y - 1):
            rdma = pltpu.make_async_remote_copy(
                src_ref=comm_ref.at[h],
                dst_ref=comm_ref.at[h + 1],
                send_sem=send_sems.at[h],
                recv_sem=recv_sems.at[h],
                device_id=(my_x, right, my_z),
                device_id_type=pl.DeviceIdType.MESH,
            )
            rdma.start()
            rdma.wait()

        slots = [comm_ref[s] for s in range(Y)]
        ms = [a[:, :, D:2 * D] for a in slots]
        m_g = jnp.maximum(jnp.maximum(ms[0], ms[1]),
                          jnp.maximum(ms[2], ms[3]))
        acc_g = jnp.zeros((B, H, D), jnp.float32)
        l_g = jnp.zeros((B, H, D), jnp.float32)
        for a, mm in zip(slots, ms):
            sc = jnp.exp(mm - m_g)
            acc_g = acc_g + a[:, :, 0:D] * sc
            l_g = l_g + a[:, :, 2 * D:3 * D] * sc
        res = acc_g / l_g
        out_ref[:] = jnp.concatenate([res, m_g, l_g], axis=-1)

    return pl.pallas_call(
        body,
        out_shape=jax.ShapeDtypeStruct((B, H, PAY), jnp.float32),
        in_specs=[pl.BlockSpec(memory_space=pltpu.VMEM)],
        out_specs=pl.BlockSpec(memory_space=pltpu.VMEM),
        scratch_shapes=[
            pltpu.VMEM((Y, B, H, PAY), jnp.float32),
            pltpu.SemaphoreType.DMA((Y - 1,)),
            pltpu.SemaphoreType.DMA((Y - 1,)),
        ],
        compiler_params=pltpu.CompilerParams(collective_id=0),
    )(packed)


def kernel(Q, K, V, bt, lens):
    combined = _ring_combine_kernel(_partial_kernel(Q, K, V, bt, lens))
    return combined[:, :, 0:D][:, None, :, :]
